# Initial kernel scaffold; baseline (speedup 1.0000x reference)
import jax
import jax.numpy as jnp
from jax.experimental import pallas as pl


def kernel(x, edge_index, batch, w_conv1, att_src1, att_dst1, b_conv1, w_skip1, b_skip1, w_conv2, att_src2, att_dst2, b_conv2, w_skip2, b_skip2, w_conv3, att_src3, att_dst3, b_conv3, w_skip3, b_skip3):
    raise NotImplementedError("write your pallas kernel here")


# speedup vs baseline: 25.0678x; 25.0678x over previous
import functools

import jax
import jax.numpy as jnp
from jax import lax
from jax.experimental import pallas as pl
from jax.experimental.pallas import tpu as pltpu
from jax.experimental.pallas import tpu_sc as plsc

N = 10000
E = 320000
F = 128
NP = 10240
NW = 32
CPW = F // NW
EW = E // NW
CHA = 2000
CHB = 8000
BN = 2048


def _tc_pre_body(prev_ref, wt_ref, at_ref, wst_ref, h_ref, a_ref, s_ref):
    p = prev_ref[...]
    h = jnp.dot(wt_ref[...], p, preferred_element_type=jnp.float32)
    h_ref[...] = h
    a_ref[...] = jnp.dot(at_ref[...], h, preferred_element_type=jnp.float32)
    s_ref[...] = jnp.dot(wst_ref[...], p, preferred_element_type=jnp.float32)


def _tc_pre(prevT, WT, AsadT, WsT):
    return pl.pallas_call(
        _tc_pre_body,
        grid=(NP // BN,),
        in_specs=[pl.BlockSpec((F, BN), lambda j: (0, j)),
                  pl.BlockSpec((F, F), lambda j: (0, 0)),
                  pl.BlockSpec((8, F), lambda j: (0, 0)),
                  pl.BlockSpec((F, F), lambda j: (0, 0))],
        out_specs=[pl.BlockSpec((F, BN), lambda j: (0, j)),
                   pl.BlockSpec((8, BN), lambda j: (0, j)),
                   pl.BlockSpec((F, BN), lambda j: (0, j))],
        out_shape=[jax.ShapeDtypeStruct((F, NP), jnp.float32),
                   jax.ShapeDtypeStruct((8, NP), jnp.float32),
                   jax.ShapeDtypeStruct((F, NP), jnp.float32)],
    )(prevT, WT, AsadT, WsT)


def _tc_fin_body(raw_ref, den_ref, skip_ref, b_ref, r_ref, o_ref, *, act):
    den = jnp.dot(r_ref[...], den_ref[...], preferred_element_type=jnp.float32)
    v = raw_ref[...] / (den + 1e-16) + b_ref[...] + skip_ref[...]
    if act == "relu":
        o_ref[...] = jnp.maximum(v, 0.0)
    else:
        o_ref[...] = 1.0 / (1.0 + jnp.exp(-v))


def _tc_fin(rawT, den8, skipT, b2, R, act):
    return pl.pallas_call(
        functools.partial(_tc_fin_body, act=act),
        grid=(NP // BN,),
        in_specs=[pl.BlockSpec((F, BN), lambda j: (0, j)),
                  pl.BlockSpec((8, BN), lambda j: (0, j)),
                  pl.BlockSpec((F, BN), lambda j: (0, j)),
                  pl.BlockSpec((F, 1), lambda j: (0, 0)),
                  pl.BlockSpec((F, 8), lambda j: (0, 0))],
        out_specs=pl.BlockSpec((F, BN), lambda j: (0, j)),
        out_shape=jax.ShapeDtypeStruct((F, NP), jnp.float32),
    )(rawT, den8, skipT, b2, R)


def _sc_phase_a(asad_flat, srcv, dstv, hd_n):
    mesh = plsc.VectorSubcoreMesh(core_axis_name="c", subcore_axis_name="s")

    @functools.partial(
        pl.kernel,
        out_type=jax.ShapeDtypeStruct((hd_n * E,), jnp.float32),
        mesh=mesh,
        compiler_params=pltpu.CompilerParams(needs_layout_passes=False),
        scratch_types=[
            pltpu.VMEM((2 * hd_n * NP,), jnp.float32),
            pltpu.VMEM((CHA,), jnp.int32),
            pltpu.VMEM((CHA,), jnp.int32),
            pltpu.VMEM((hd_n * CHA,), jnp.float32),
        ],
    )
    def ka(asad_hbm, src_hbm, dst_hbm, ex_hbm, tabs, srcb, dstb, exb):
        wid = lax.axis_index("s") * 2 + lax.axis_index("c")
        base = wid * EW
        pltpu.sync_copy(asad_hbm.at[pl.ds(0, hd_n * NP)],
                        tabs.at[pl.ds(0, hd_n * NP)])
        pltpu.sync_copy(asad_hbm.at[pl.ds(4 * NP, hd_n * NP)],
                        tabs.at[pl.ds(hd_n * NP, hd_n * NP)])

        def chunk(ci, carry):
            cb = base + ci * CHA
            pltpu.sync_copy(src_hbm.at[pl.ds(cb, CHA)], srcb)
            pltpu.sync_copy(dst_hbm.at[pl.ds(cb, CHA)], dstb)

            def batch(j, c2):
                s16 = srcb[pl.ds(j * 16, 16)]
                d16 = dstb[pl.ds(j * 16, 16)]
                for hd in range(hd_n):
                    av = plsc.load_gather(tabs, [s16 + hd * NP])
                    dv = plsc.load_gather(tabs, [d16 + (hd_n + hd) * NP])
                    a = av + dv
                    a = jnp.where(a >= 0.0, a, a * 0.2)
                    exb[pl.ds(hd * CHA + j * 16, 16)] = jnp.exp(a)
                return c2

            lax.fori_loop(0, CHA // 16, batch, 0)
            for hd in range(hd_n):
                pltpu.sync_copy(exb.at[pl.ds(hd * CHA, CHA)],
                                ex_hbm.at[pl.ds(hd * E + cb, CHA)])
            return carry

        lax.fori_loop(0, EW // CHA, chunk, 0)

    return ka(asad_flat, srcv, dstv)


def _sc_phase_b(h_flat, srcv, dstv, ex_flat, hd_n):
    mesh = plsc.VectorSubcoreMesh(core_axis_name="c", subcore_axis_name="s")
    wper = NW // hd_n

    @functools.partial(
        pl.kernel,
        out_type=(jax.ShapeDtypeStruct((F * NP,), jnp.float32),
                  jax.ShapeDtypeStruct((hd_n * NP,), jnp.float32)),
        mesh=mesh,
        compiler_params=pltpu.CompilerParams(needs_layout_passes=False),
        scratch_types=[
            pltpu.VMEM((CPW * NP,), jnp.float32),
            pltpu.VMEM((CPW * NP,), jnp.float32),
            pltpu.VMEM((NP,), jnp.float32),
            pltpu.VMEM((CHB,), jnp.int32),
            pltpu.VMEM((CHB,), jnp.int32),
            pltpu.VMEM((CHB,), jnp.float32),
        ],
    )
    def kb(h_hbm, src_hbm, dst_hbm, ex_hbm, out_hbm, den_hbm,
           hrows, orows, denrow, srcb, dstb, exb):
        wid = lax.axis_index("s") * 2 + lax.axis_index("c")
        hd = wid // wper
        c0 = wid * CPW
        is_den = (wid % wper) == 0
        pltpu.sync_copy(h_hbm.at[pl.ds(c0 * NP, CPW * NP)], hrows)

        def z(i, c):
            zz = jnp.zeros((16,), jnp.float32)
            orows[pl.ds(i * 16, 16)] = zz
            return c

        lax.fori_loop(0, CPW * NP // 16, z, 0)

        def zd(i, c):
            denrow[pl.ds(i * 16, 16)] = jnp.zeros((16,), jnp.float32)
            return c

        lax.fori_loop(0, NP // 16, zd, 0)

        def chunk(ci, carry):
            cb = ci * CHB
            pltpu.sync_copy(src_hbm.at[pl.ds(cb, CHB)], srcb)
            pltpu.sync_copy(dst_hbm.at[pl.ds(cb, CHB)], dstb)
            pltpu.sync_copy(ex_hbm.at[pl.ds(hd * E + cb, CHB)], exb)

            def batch(j, c2):
                s16 = srcb[pl.ds(j * 16, 16)]
                d16 = dstb[pl.ds(j * 16, 16)]
                ev = exb[pl.ds(j * 16, 16)]
                for cc in range(CPW):
                    hv = plsc.load_gather(hrows, [s16 + cc * NP])
                    plsc.addupdate_scatter(orows, [d16 + cc * NP], hv * ev)

                @pl.when(is_den)
                def _():
                    plsc.addupdate_scatter(denrow, [d16], ev)
                return c2

            lax.fori_loop(0, CHB // 16, batch, 0)
            return carry

        lax.fori_loop(0, E // CHB, chunk, 0)
        pltpu.sync_copy(orows, out_hbm.at[pl.ds(c0 * NP, CPW * NP)])

        @pl.when(is_den)
        def _():
            pltpu.sync_copy(denrow, den_hbm.at[pl.ds(hd * NP, NP)])

    return kb(h_flat, srcv, dstv, ex_flat)


def _asad(a_src, a_dst):
    hd_n, c = a_src.shape
    A = jnp.zeros((F, 8), jnp.float32)
    for hd in range(hd_n):
        A = A.at[hd * c:(hd + 1) * c, hd].set(a_src[hd])
        A = A.at[hd * c:(hd + 1) * c, 4 + hd].set(a_dst[hd])
    return A.T


def _layer(prevT, srcv, dstv, W, a_src, a_dst, b, Wskip, bskip, act):
    hd_n = a_src.shape[0]
    hT, asadT, skipT = _tc_pre(prevT, W.T, _asad(a_src, a_dst), Wskip.T)
    ex_flat = _sc_phase_a(asadT.reshape(-1), srcv, dstv, hd_n)
    out_flat, den_flat = _sc_phase_b(hT.reshape(-1), srcv, dstv, ex_flat, hd_n)
    outT = out_flat.reshape(F, NP)
    denT = den_flat.reshape(hd_n, NP)
    den8 = jnp.zeros((8, NP), jnp.float32).at[:hd_n].set(denT)
    R = (jnp.arange(8)[None, :] ==
         (jnp.arange(F)[:, None] * hd_n // F)).astype(jnp.float32)
    b2 = (b + bskip)[:, None]
    return _tc_fin(outT, den8, skipT, b2, R, act)


def kernel(x, edge_index, batch, w_conv1, att_src1, att_dst1, b_conv1,
           w_skip1, b_skip1, w_conv2, att_src2, att_dst2, b_conv2,
           w_skip2, b_skip2, w_conv3, att_src3, att_dst3, b_conv3,
           w_skip3, b_skip3):
    srcv = edge_index[0]
    dstv = edge_index[1]
    xT = jnp.zeros((F, NP), jnp.float32).at[:, :N].set(x.T)
    h1 = _layer(xT, srcv, dstv, w_conv1, att_src1, att_dst1, b_conv1,
                w_skip1, b_skip1, "relu")
    h2 = _layer(h1, srcv, dstv, w_conv2, att_src2, att_dst2, b_conv2,
                w_skip2, b_skip2, "relu")
    h3 = _layer(h2, srcv, dstv, w_conv3, att_src3, att_dst3, b_conv3,
                w_skip3, b_skip3, "sigmoid")
    return h3[:, :N].T


# speedup vs baseline: 25.3010x; 1.0093x over previous
import functools

import jax
import jax.numpy as jnp
from jax import lax
from jax.experimental import pallas as pl
from jax.experimental.pallas import tpu as pltpu
from jax.experimental.pallas import tpu_sc as plsc

N = 10000
E = 320000
F = 128
NP = 10240
NW = 32
CPW = F // NW
EW = E // NW
CHA = 2000
CHB = 8000
BN = 2048


def _tc_pre_body(prev_ref, wt_ref, at_ref, wst_ref, h_ref, a_ref, s_ref):
    p = prev_ref[...]
    h = jnp.dot(wt_ref[...], p, preferred_element_type=jnp.float32)
    h_ref[...] = h
    a_ref[...] = jnp.dot(at_ref[...], h, preferred_element_type=jnp.float32)
    s_ref[...] = jnp.dot(wst_ref[...], p, preferred_element_type=jnp.float32)


def _tc_pre(prevT, WT, AsadT, WsT):
    return pl.pallas_call(
        _tc_pre_body,
        grid=(NP // BN,),
        in_specs=[pl.BlockSpec((F, BN), lambda j: (0, j)),
                  pl.BlockSpec((F, F), lambda j: (0, 0)),
                  pl.BlockSpec((8, F), lambda j: (0, 0)),
                  pl.BlockSpec((F, F), lambda j: (0, 0))],
        out_specs=[pl.BlockSpec((F, BN), lambda j: (0, j)),
                   pl.BlockSpec((8, BN), lambda j: (0, j)),
                   pl.BlockSpec((F, BN), lambda j: (0, j))],
        out_shape=[jax.ShapeDtypeStruct((F, NP), jnp.float32),
                   jax.ShapeDtypeStruct((8, NP), jnp.float32),
                   jax.ShapeDtypeStruct((F, NP), jnp.float32)],
    )(prevT, WT, AsadT, WsT)


def _tc_fin_body(raw_ref, den_ref, skip_ref, b_ref, r_ref, o_ref, *, act):
    den = jnp.dot(r_ref[...], den_ref[...], preferred_element_type=jnp.float32)
    v = raw_ref[...] / (den + 1e-16) + b_ref[...] + skip_ref[...]
    if act == "relu":
        o_ref[...] = jnp.maximum(v, 0.0)
    else:
        o_ref[...] = 1.0 / (1.0 + jnp.exp(-v))


def _tc_fin(rawT, den8, skipT, b2, R, act):
    return pl.pallas_call(
        functools.partial(_tc_fin_body, act=act),
        grid=(NP // BN,),
        in_specs=[pl.BlockSpec((F, BN), lambda j: (0, j)),
                  pl.BlockSpec((8, BN), lambda j: (0, j)),
                  pl.BlockSpec((F, BN), lambda j: (0, j)),
                  pl.BlockSpec((F, 1), lambda j: (0, 0)),
                  pl.BlockSpec((F, 8), lambda j: (0, 0))],
        out_specs=pl.BlockSpec((F, BN), lambda j: (0, j)),
        out_shape=jax.ShapeDtypeStruct((F, NP), jnp.float32),
    )(rawT, den8, skipT, b2, R)


def _sc_phase_a(asad_flat, srcv, dstv, hd_n):
    mesh = plsc.VectorSubcoreMesh(core_axis_name="c", subcore_axis_name="s")

    @functools.partial(
        pl.kernel,
        out_type=jax.ShapeDtypeStruct((hd_n * E,), jnp.float32),
        mesh=mesh,
        compiler_params=pltpu.CompilerParams(needs_layout_passes=False),
        scratch_types=[
            pltpu.VMEM((2 * hd_n * NP,), jnp.float32),
            pltpu.VMEM((CHA,), jnp.int32),
            pltpu.VMEM((CHA,), jnp.int32),
            pltpu.VMEM((hd_n * CHA,), jnp.float32),
        ],
    )
    def ka(asad_hbm, src_hbm, dst_hbm, ex_hbm, tabs, srcb, dstb, exb):
        wid = lax.axis_index("s") * 2 + lax.axis_index("c")
        base = wid * EW
        pltpu.sync_copy(asad_hbm.at[pl.ds(0, hd_n * NP)],
                        tabs.at[pl.ds(0, hd_n * NP)])
        pltpu.sync_copy(asad_hbm.at[pl.ds(4 * NP, hd_n * NP)],
                        tabs.at[pl.ds(hd_n * NP, hd_n * NP)])

        def chunk(ci, carry):
            cb = base + ci * CHA
            pltpu.sync_copy(src_hbm.at[pl.ds(cb, CHA)], srcb)
            pltpu.sync_copy(dst_hbm.at[pl.ds(cb, CHA)], dstb)

            def batch(j, c2):
                s16 = srcb[pl.ds(j * 16, 16)]
                d16 = dstb[pl.ds(j * 16, 16)]
                for hd in range(hd_n):
                    av = plsc.load_gather(tabs, [s16 + hd * NP])
                    dv = plsc.load_gather(tabs, [d16 + (hd_n + hd) * NP])
                    a = av + dv
                    a = jnp.where(a >= 0.0, a, a * 0.2)
                    exb[pl.ds(hd * CHA + j * 16, 16)] = jnp.exp(a)
                return c2

            lax.fori_loop(0, CHA // 16, batch, 0)
            for hd in range(hd_n):
                pltpu.sync_copy(exb.at[pl.ds(hd * CHA, CHA)],
                                ex_hbm.at[pl.ds(hd * E + cb, CHA)])
            return carry

        lax.fori_loop(0, EW // CHA, chunk, 0)

    return ka(asad_flat, srcv, dstv)


def _sc_phase_b(h_flat, srcv, dstv, ex_flat, hd_n):
    mesh = plsc.VectorSubcoreMesh(core_axis_name="c", subcore_axis_name="s")
    wper = NW // hd_n

    @functools.partial(
        pl.kernel,
        out_type=(jax.ShapeDtypeStruct((F * NP,), jnp.float32),
                  jax.ShapeDtypeStruct((hd_n * NP,), jnp.float32)),
        mesh=mesh,
        compiler_params=pltpu.CompilerParams(needs_layout_passes=False),
        scratch_types=[
            pltpu.VMEM((CPW * NP,), jnp.float32),
            pltpu.VMEM((CPW * NP,), jnp.float32),
            pltpu.VMEM((NP,), jnp.float32),
            pltpu.VMEM((CHB,), jnp.int32),
            pltpu.VMEM((CHB,), jnp.int32),
            pltpu.VMEM((CHB,), jnp.float32),
        ],
    )
    def kb(h_hbm, src_hbm, dst_hbm, ex_hbm, out_hbm, den_hbm,
           hrows, orows, denrow, srcb, dstb, exb):
        wid = lax.axis_index("s") * 2 + lax.axis_index("c")
        hd = wid // wper
        c0 = wid * CPW
        is_den = (wid % wper) == 0
        pltpu.sync_copy(h_hbm.at[pl.ds(c0 * NP, CPW * NP)], hrows)

        def z(i, c):
            zz = jnp.zeros((16,), jnp.float32)
            orows[pl.ds(i * 16, 16)] = zz
            return c

        lax.fori_loop(0, CPW * NP // 16, z, 0)

        def zd(i, c):
            denrow[pl.ds(i * 16, 16)] = jnp.zeros((16,), jnp.float32)
            return c

        lax.fori_loop(0, NP // 16, zd, 0)

        def chunk(ci, carry):
            cb = ci * CHB
            pltpu.sync_copy(src_hbm.at[pl.ds(cb, CHB)], srcb)
            pltpu.sync_copy(dst_hbm.at[pl.ds(cb, CHB)], dstb)
            pltpu.sync_copy(ex_hbm.at[pl.ds(hd * E + cb, CHB)], exb)

            def batch(j, c2):
                for u in range(4):
                    off = j * 64 + u * 16
                    s16 = srcb[pl.ds(off, 16)]
                    d16 = dstb[pl.ds(off, 16)]
                    ev = exb[pl.ds(off, 16)]
                    for cc in range(CPW):
                        hv = plsc.load_gather(hrows, [s16 + cc * NP])
                        plsc.addupdate_scatter(orows, [d16 + cc * NP], hv * ev)

                    @pl.when(is_den)
                    def _():
                        plsc.addupdate_scatter(denrow, [d16], ev)
                return c2

            lax.fori_loop(0, CHB // 64, batch, 0)
            return carry

        lax.fori_loop(0, E // CHB, chunk, 0)
        pltpu.sync_copy(orows, out_hbm.at[pl.ds(c0 * NP, CPW * NP)])

        @pl.when(is_den)
        def _():
            pltpu.sync_copy(denrow, den_hbm.at[pl.ds(hd * NP, NP)])

    return kb(h_flat, srcv, dstv, ex_flat)


def _asad(a_src, a_dst):
    hd_n, c = a_src.shape
    A = jnp.zeros((F, 8), jnp.float32)
    for hd in range(hd_n):
        A = A.at[hd * c:(hd + 1) * c, hd].set(a_src[hd])
        A = A.at[hd * c:(hd + 1) * c, 4 + hd].set(a_dst[hd])
    return A.T


def _layer(prevT, srcv, dstv, W, a_src, a_dst, b, Wskip, bskip, act):
    hd_n = a_src.shape[0]
    hT, asadT, skipT = _tc_pre(prevT, W.T, _asad(a_src, a_dst), Wskip.T)
    ex_flat = _sc_phase_a(asadT.reshape(-1), srcv, dstv, hd_n)
    out_flat, den_flat = _sc_phase_b(hT.reshape(-1), srcv, dstv, ex_flat, hd_n)
    outT = out_flat.reshape(F, NP)
    denT = den_flat.reshape(hd_n, NP)
    den8 = jnp.zeros((8, NP), jnp.float32).at[:hd_n].set(denT)
    R = (jnp.arange(8)[None, :] ==
         (jnp.arange(F)[:, None] * hd_n // F)).astype(jnp.float32)
    b2 = (b + bskip)[:, None]
    return _tc_fin(outT, den8, skipT, b2, R, act)


def kernel(x, edge_index, batch, w_conv1, att_src1, att_dst1, b_conv1,
           w_skip1, b_skip1, w_conv2, att_src2, att_dst2, b_conv2,
           w_skip2, b_skip2, w_conv3, att_src3, att_dst3, b_conv3,
           w_skip3, b_skip3):
    srcv = edge_index[0]
    dstv = edge_index[1]
    xT = jnp.zeros((F, NP), jnp.float32).at[:, :N].set(x.T)
    h1 = _layer(xT, srcv, dstv, w_conv1, att_src1, att_dst1, b_conv1,
                w_skip1, b_skip1, "relu")
    h2 = _layer(h1, srcv, dstv, w_conv2, att_src2, att_dst2, b_conv2,
                w_skip2, b_skip2, "relu")
    h3 = _layer(h2, srcv, dstv, w_conv3, att_src3, att_dst3, b_conv3,
                w_skip3, b_skip3, "sigmoid")
    return h3[:, :N].T
